# baseline (speedup 1.0000x reference)
import functools
import jax
import jax.numpy as jnp
from jax import lax
from jax.experimental import pallas as pl
from jax.experimental.pallas import tpu as pltpu
from jax.experimental.pallas import tpu_sc as plsc

BATCH = 16384
NUM_FIELDS = 26
EMBED_DIM = 16
FIELD_SIZE = 100000
NUM_EMB = NUM_FIELDS * FIELD_SIZE

NC, NS = 2, 16
NW = NC * NS
NBLK = NUM_FIELDS * (BATCH // 128)
BLK_W = NBLK // NW
NCB = (NUM_EMB + 127) // 128
NGRAN = NCB * 16 * 8


def _reblock_body(t_ref, o_ref):
    o_ref[...] = t_ref[...].reshape(1, EMBED_DIM, 128)


_reblock = pl.pallas_call(
    _reblock_body,
    grid=(NCB,),
    in_specs=[pl.BlockSpec((EMBED_DIM, 128), lambda i: (0, i))],
    out_specs=pl.BlockSpec((1, EMBED_DIM, 128), lambda i: (i, 0, 0)),
    out_shape=jax.ShapeDtypeStruct((NCB, EMBED_DIM, 128), jnp.float32),
)

_mesh = plsc.VectorSubcoreMesh(core_axis_name="c", subcore_axis_name="s")


@functools.partial(
    pl.kernel,
    out_type=jax.ShapeDtypeStruct((NUM_FIELDS, 2, 128, 8, 128), jnp.float32),
    mesh=_mesh,
    scratch_types=[
        pltpu.VMEM((BLK_W, 128), jnp.int32),
        pltpu.VMEM((BLK_W, 128), jnp.int32),
        pltpu.VMEM((EMBED_DIM, 128), jnp.int32),
        pltpu.VMEM((2, EMBED_DIM, 128, 16), jnp.float32),
        pltpu.VMEM((2, 2, 8, 128), jnp.float32),
        pltpu.SemaphoreType.DMA,
        pltpu.SemaphoreType.DMA,
    ],
    compiler_params=pltpu.CompilerParams(
        use_tc_tiling_on_sc=False, needs_layout_passes=False
    ),
)
def _emb_lookup(x_hbm, tg_hbm, out_hbm, idx_v, sub_v, hrow_v, stg_v, tile_v,
                gsem, wsem):
    wid = lax.axis_index("s") * NC + lax.axis_index("c")
    g0 = wid * BLK_W

    pltpu.sync_copy(x_hbm.at[pl.ds(g0, BLK_W)], idx_v)

    def prep(r, carry):
        off = FIELD_SIZE * ((g0 + r) // 128)
        for c in range(8):
            sl = pl.ds(c * 16, 16)
            full = idx_v[r, sl] + off
            idx_v[r, sl] = ((full >> 7) << 7) + ((full & 127) >> 4)
            sub_v[r, sl] = full & 15
        return carry

    lax.fori_loop(0, BLK_W, prep, 0)

    bvec = lax.broadcasted_iota(jnp.int32, (16,), 0)

    def gather_of(r, d):
        return pltpu.make_async_copy(
            tg_hbm.at[hrow_v.at[d]], stg_v.at[r & 1, d], gsem
        )

    def write_of(r, dg):
        g = g0 + r
        return pltpu.make_async_copy(
            tile_v.at[r & 1, dg], out_hbm.at[g // 128, dg, g % 128], wsem
        )

    def fire(r):
        for d in range(EMBED_DIM):
            for c in range(8):
                sl = pl.ds(c * 16, 16)
                hrow_v[d, sl] = idx_v[r, sl] + (8 * d)
        for d in range(EMBED_DIM):
            gather_of(r, d).start()

    fire(0)

    def body(r, carry):
        @pl.when(r >= 2)
        def _():
            write_of(r - 2, 0).wait()
            write_of(r - 2, 1).wait()

        for d in range(EMBED_DIM):
            gather_of(r, d).wait()

        @pl.when(r + 1 < BLK_W)
        def _():
            fire(r + 1)

        buf = r & 1
        for k in range(8):
            row = bvec + k * 16
            sv = sub_v[r, pl.ds(k * 16, 16)]
            for d in range(EMBED_DIM):
                val = plsc.load_gather(stg_v.at[buf, d], [row, sv])
                tile_v[buf, d // 8, d % 8, pl.ds(k * 16, 16)] = val

        write_of(r, 0).start()
        write_of(r, 1).start()
        return carry

    lax.fori_loop(0, BLK_W, body, 0)
    write_of(BLK_W - 2, 0).wait()
    write_of(BLK_W - 2, 1).wait()
    write_of(BLK_W - 1, 0).wait()
    write_of(BLK_W - 1, 1).wait()


def kernel(x, table):
    tt = table.T
    tg = _reblock(tt).reshape(NGRAN, 16)
    x2 = x.T.reshape(NBLK, 128)
    out5 = _emb_lookup(x2, tg)
    return out5.transpose(2, 4, 0, 1, 3).reshape(BATCH, NUM_FIELDS, EMBED_DIM)


# speedup vs baseline: 16.0582x; 16.0582x over previous
import functools
import jax
import jax.numpy as jnp
from jax import lax
from jax.experimental import pallas as pl
from jax.experimental.pallas import tpu as pltpu
from jax.experimental.pallas import tpu_sc as plsc

BATCH = 16384
NUM_FIELDS = 26
EMBED_DIM = 16
FIELD_SIZE = 100000
NUM_EMB = NUM_FIELDS * FIELD_SIZE

NC, NS = 2, 16
NW = NC * NS
NBLK = NUM_FIELDS * (BATCH // 128)
BLK_W = NBLK // NW
NCB = (NUM_EMB + 127) // 128
NGRAN = NCB * 16 * 8


_mesh = plsc.VectorSubcoreMesh(core_axis_name="c", subcore_axis_name="s")

NFULL = NCB - 1
CB_BASE = NFULL // NW
CB_EXTRA = NFULL - CB_BASE * NW


@functools.partial(
    pl.kernel,
    out_type=jax.ShapeDtypeStruct((NCB, EMBED_DIM, 128), jnp.float32),
    mesh=_mesh,
    scratch_types=[
        pltpu.VMEM((4, 2, 8, 128), jnp.float32),
        pltpu.SemaphoreType.DMA,
        pltpu.SemaphoreType.DMA,
    ],
    compiler_params=pltpu.CompilerParams(
        use_tc_tiling_on_sc=True, needs_layout_passes=False
    ),
)
def _reblock(tt_hbm, tail_hbm, out_hbm, st_v, isem, osem):
    wid = lax.axis_index("s") * NC + lax.axis_index("c")
    cb0 = wid * CB_BASE + jnp.minimum(wid, CB_EXTRA)
    nch = CB_BASE + (wid < CB_EXTRA).astype(jnp.int32)

    def in_of(i, dg):
        return pltpu.make_async_copy(
            tt_hbm.at[pl.ds(dg * 8, 8), pl.ds((cb0 + i) * 128, 128)],
            st_v.at[i & 3, dg],
            isem,
        )

    def out_of(i, dg):
        return pltpu.make_async_copy(
            st_v.at[i & 3, dg],
            out_hbm.at[cb0 + i, pl.ds(dg * 8, 8)],
            osem,
        )

    for i in range(2):
        in_of(i, 0).start()
        in_of(i, 1).start()

    def body(i, carry):
        @pl.when(i >= 2)
        def _():
            out_of(i - 2, 0).wait()
            out_of(i - 2, 1).wait()

        @pl.when(i + 2 < nch)
        def _():
            in_of(i + 2, 0).start()
            in_of(i + 2, 1).start()

        in_of(i, 0).wait()
        in_of(i, 1).wait()
        out_of(i, 0).start()
        out_of(i, 1).start()
        return carry

    lax.fori_loop(0, nch, body, 0)
    out_of(nch - 2, 0).wait()
    out_of(nch - 2, 1).wait()
    out_of(nch - 1, 0).wait()
    out_of(nch - 1, 1).wait()

    @pl.when(wid == NW - 1)
    def _():
        for dg in range(2):
            pltpu.sync_copy(tail_hbm.at[pl.ds(dg * 8, 8)], st_v.at[0, dg])
            pltpu.sync_copy(st_v.at[0, dg], out_hbm.at[NCB - 1, pl.ds(dg * 8, 8)])


@functools.partial(
    pl.kernel,
    out_type=jax.ShapeDtypeStruct((NUM_FIELDS, 2, 128, 8, 128), jnp.float32),
    mesh=_mesh,
    scratch_types=[
        pltpu.VMEM((BLK_W, 128), jnp.int32),
        pltpu.VMEM((BLK_W, 128), jnp.int32),
        pltpu.VMEM((EMBED_DIM, 128), jnp.int32),
        pltpu.VMEM((2, EMBED_DIM, 128, 16), jnp.float32),
        pltpu.VMEM((2, 2, 8, 128), jnp.float32),
        pltpu.SemaphoreType.DMA,
        pltpu.SemaphoreType.DMA,
    ],
    compiler_params=pltpu.CompilerParams(
        use_tc_tiling_on_sc=False, needs_layout_passes=False
    ),
)
def _emb_lookup(x_hbm, tg_hbm, out_hbm, idx_v, sub_v, hrow_v, stg_v, tile_v,
                gsem, wsem):
    wid = lax.axis_index("s") * NC + lax.axis_index("c")
    g0 = wid * BLK_W

    pltpu.sync_copy(x_hbm.at[pl.ds(g0, BLK_W)], idx_v)

    def prep(r, carry):
        off = FIELD_SIZE * ((g0 + r) // 128)
        for c in range(8):
            sl = pl.ds(c * 16, 16)
            full = idx_v[r, sl] + off
            idx_v[r, sl] = ((full >> 7) << 7) + ((full & 127) >> 4)
            sub_v[r, sl] = full & 15
        return carry

    lax.fori_loop(0, BLK_W, prep, 0)

    bvec = lax.broadcasted_iota(jnp.int32, (16,), 0)

    def gather_of(r, d):
        return pltpu.make_async_copy(
            tg_hbm.at[hrow_v.at[d]], stg_v.at[r & 1, d], gsem
        )

    def write_of(r, dg):
        g = g0 + r
        return pltpu.make_async_copy(
            tile_v.at[r & 1, dg], out_hbm.at[g // 128, dg, g % 128], wsem
        )

    def fire(r):
        for d in range(EMBED_DIM):
            for c in range(8):
                sl = pl.ds(c * 16, 16)
                hrow_v[d, sl] = idx_v[r, sl] + (8 * d)
        for d in range(EMBED_DIM):
            gather_of(r, d).start()

    fire(0)

    def body(r, carry):
        @pl.when(r >= 2)
        def _():
            write_of(r - 2, 0).wait()
            write_of(r - 2, 1).wait()

        for d in range(EMBED_DIM):
            gather_of(r, d).wait()

        @pl.when(r + 1 < BLK_W)
        def _():
            fire(r + 1)

        buf = r & 1
        for k in range(8):
            row = bvec + k * 16
            sv = sub_v[r, pl.ds(k * 16, 16)]
            for d in range(EMBED_DIM):
                val = plsc.load_gather(stg_v.at[buf, d], [row, sv])
                tile_v[buf, d // 8, d % 8, pl.ds(k * 16, 16)] = val

        write_of(r, 0).start()
        write_of(r, 1).start()
        return carry

    lax.fori_loop(0, BLK_W, body, 0)
    write_of(BLK_W - 2, 0).wait()
    write_of(BLK_W - 2, 1).wait()
    write_of(BLK_W - 1, 0).wait()
    write_of(BLK_W - 1, 1).wait()


def kernel(x, table):
    tt = table.T
    tail = jnp.pad(tt[:, NFULL * 128:], ((0, 0), (0, NCB * 128 - NUM_EMB)))
    tg = _reblock(tt, tail).reshape(NGRAN, 16)
    x2 = x.T.reshape(NBLK, 128)
    out5 = _emb_lookup(x2, tg)
    return out5.transpose(2, 4, 0, 1, 3).reshape(BATCH, NUM_FIELDS, EMBED_DIM)
